# baseline (speedup 1.0000x reference)
import functools

import jax
import jax.numpy as jnp
from jax import lax
from jax.experimental import pallas as pl
from jax.experimental.pallas import tpu as pltpu
from jax.experimental.pallas import tpu_sc as plsc

G = 256

_TM1 = 400
_TM2 = 2000


def _pass1_body(scale_ref, adj_ref, hb_ref, ht_ref, w1t_ref, b1_ref,
                t_ref, s1_ref, s2_ref):
    i = pl.program_id(0)
    acc = jnp.dot(adj_ref[...].astype(jnp.bfloat16), hb_ref[...],
                  preferred_element_type=jnp.float32)
    out = scale_ref[...] * ht_ref[...] + acc
    t = jnp.dot(out, w1t_ref[...], preferred_element_type=jnp.float32)
    t = t + b1_ref[...]
    t_ref[...] = t

    @pl.when(i == 0)
    def _():
        s1_ref[...] = jnp.zeros_like(s1_ref)
        s2_ref[...] = jnp.zeros_like(s2_ref)

    s1_ref[...] += jnp.sum(t, axis=0, keepdims=True)
    s2_ref[...] += jnp.sum(t * t, axis=0, keepdims=True)


def _pass1(h32, hb16, adj, scale, w1t, b1row):
    n, hd = h32.shape
    return pl.pallas_call(
        _pass1_body,
        grid=(n // _TM1,),
        in_specs=[
            pl.BlockSpec((1, 1), lambda i: (0, 0)),
            pl.BlockSpec((_TM1, adj.shape[1]), lambda i: (i, 0)),
            pl.BlockSpec((n, hd), lambda i: (0, 0)),
            pl.BlockSpec((_TM1, hd), lambda i: (i, 0)),
            pl.BlockSpec((hd, hd), lambda i: (0, 0)),
            pl.BlockSpec((1, hd), lambda i: (0, 0)),
        ],
        out_specs=[
            pl.BlockSpec((_TM1, hd), lambda i: (i, 0)),
            pl.BlockSpec((1, hd), lambda i: (0, 0)),
            pl.BlockSpec((1, hd), lambda i: (0, 0)),
        ],
        out_shape=[
            jax.ShapeDtypeStruct((n, hd), jnp.float32),
            jax.ShapeDtypeStruct((1, hd), jnp.float32),
            jax.ShapeDtypeStruct((1, hd), jnp.float32),
        ],
    )(scale, adj, hb16, h32, w1t, b1row)


def _bn_relu(v_ref, s1_ref, s2_ref, g_ref, be_ref, n):
    mean = s1_ref[...] * (1.0 / n)
    var = s2_ref[...] * (1.0 / n) - mean * mean
    rstd = lax.rsqrt(var + 1e-5)
    return jnp.maximum(g_ref[...] * (v_ref[...] - mean) * rstd + be_ref[...],
                       0.0)


def _pass2_body(t_ref, s1_ref, s2_ref, g_ref, be_ref, w2t_ref, b2_ref,
                u_ref, q1_ref, q2_ref, *, n):
    i = pl.program_id(0)
    h1 = _bn_relu(t_ref, s1_ref, s2_ref, g_ref, be_ref, n)
    u = jnp.dot(h1, w2t_ref[...], preferred_element_type=jnp.float32)
    u = u + b2_ref[...]
    u_ref[...] = u

    @pl.when(i == 0)
    def _():
        q1_ref[...] = jnp.zeros_like(q1_ref)
        q2_ref[...] = jnp.zeros_like(q2_ref)

    q1_ref[...] += jnp.sum(u, axis=0, keepdims=True)
    q2_ref[...] += jnp.sum(u * u, axis=0, keepdims=True)


def _pass2(t, s1, s2, grow, berow, w2t, b2row):
    n, hd = t.shape
    import functools
    return pl.pallas_call(
        functools.partial(_pass2_body, n=n),
        grid=(n // _TM2,),
        in_specs=[
            pl.BlockSpec((_TM2, hd), lambda i: (i, 0)),
            pl.BlockSpec((1, hd), lambda i: (0, 0)),
            pl.BlockSpec((1, hd), lambda i: (0, 0)),
            pl.BlockSpec((1, hd), lambda i: (0, 0)),
            pl.BlockSpec((1, hd), lambda i: (0, 0)),
            pl.BlockSpec((hd, hd), lambda i: (0, 0)),
            pl.BlockSpec((1, hd), lambda i: (0, 0)),
        ],
        out_specs=[
            pl.BlockSpec((_TM2, hd), lambda i: (i, 0)),
            pl.BlockSpec((1, hd), lambda i: (0, 0)),
            pl.BlockSpec((1, hd), lambda i: (0, 0)),
        ],
        out_shape=[
            jax.ShapeDtypeStruct((n, hd), jnp.float32),
            jax.ShapeDtypeStruct((1, hd), jnp.float32),
            jax.ShapeDtypeStruct((1, hd), jnp.float32),
        ],
    )(t, s1, s2, grow, berow, w2t, b2row)


def _pass3_body(u_ref, q1_ref, q2_ref, g_ref, be_ref,
                h2f_ref, h2b_ref, *, n):
    h2 = _bn_relu(u_ref, q1_ref, q2_ref, g_ref, be_ref, n)
    h2f_ref[...] = h2
    h2b_ref[...] = h2.astype(jnp.bfloat16)


def _pass3(u, q1, q2, grow, berow):
    n, hd = u.shape
    return pl.pallas_call(
        functools.partial(_pass3_body, n=n),
        grid=(n // _TM2,),
        in_specs=[
            pl.BlockSpec((_TM2, hd), lambda i: (i, 0)),
            pl.BlockSpec((1, hd), lambda i: (0, 0)),
            pl.BlockSpec((1, hd), lambda i: (0, 0)),
            pl.BlockSpec((1, hd), lambda i: (0, 0)),
            pl.BlockSpec((1, hd), lambda i: (0, 0)),
        ],
        out_specs=[
            pl.BlockSpec((_TM2, hd), lambda i: (i, 0)),
            pl.BlockSpec((_TM2, hd), lambda i: (i, 0)),
        ],
        out_shape=[
            jax.ShapeDtypeStruct((n, hd), jnp.float32),
            jax.ShapeDtypeStruct((n, hd), jnp.bfloat16),
        ],
    )(u, q1, q2, grow, berow)


_BS = 80


def _segsum_sc2(ha, hb, idx_all, zeros2):
    n, hd = ha.shape
    nblk = n // _BS
    tot = 2 * nblk
    kmax = (tot + 31) // 32
    mesh = plsc.VectorSubcoreMesh(core_axis_name="c", subcore_axis_name="s")

    @functools.partial(
        pl.kernel,
        out_type=jax.ShapeDtypeStruct((2, 2 * G, hd), jnp.float32),
        mesh=mesh,
        scratch_types=[
            pltpu.VMEM((_BS,), jnp.int32),
            pltpu.VMEM((_BS, hd), jnp.float32),
            pltpu.VMEM_SHARED((2 * G, hd), jnp.float32),
        ],
    )
    def seg_kernel(ha_hbm, hb_hbm, idx_hbm, zero_hbm, out_hbm,
                   idx_v, rows_v, shared):
        cid = lax.axis_index("c")
        sid = lax.axis_index("s")
        wid = cid * 16 + sid

        @pl.when(sid == 0)
        def _():
            pltpu.sync_copy(zero_hbm, shared)

        plsc.subcore_barrier()

        for k in range(kmax):
            b = wid + 32 * k

            @pl.when(b < tot)
            def _():
                lyr = b // nblk
                base = (b - lyr * nblk) * _BS
                pltpu.sync_copy(idx_hbm.at[pl.ds(lyr * n + base, _BS)],
                                idx_v)

                @pl.when(lyr == 0)
                def _():
                    pltpu.sync_copy(ha_hbm.at[pl.ds(base, _BS)], rows_v)

                @pl.when(lyr == 1)
                def _():
                    pltpu.sync_copy(hb_hbm.at[pl.ds(base, _BS)], rows_v)

                pltpu.sync_copy(rows_v, shared.at[idx_v], add=True)

        plsc.subcore_barrier()

        @pl.when(sid == 0)
        def _():
            pltpu.sync_copy(shared, out_hbm.at[cid])

    return seg_kernel(ha, hb, idx_all, zeros2)


def _readout_body(p_ref, wa_ref, wb_ref, b1_ref, w2_ref, b2_ref,
                  o_ref):
    acc = p_ref[0] + p_ref[1]
    seg1 = acc[:G]
    seg2 = acc[G:]
    o1 = (jnp.dot(seg1, wa_ref[...], preferred_element_type=jnp.float32)
          + jnp.dot(seg2, wb_ref[...], preferred_element_type=jnp.float32)
          + b1_ref[...])
    o1 = jnp.maximum(o1, 0.0)
    o_ref[...] = jnp.dot(o1, w2_ref[...],
                         preferred_element_type=jnp.float32) + b2_ref[...]


def _readout(p, wa, wb, b1row, w2, b2row):
    c = w2.shape[1]
    return pl.pallas_call(
        _readout_body,
        out_shape=jax.ShapeDtypeStruct((G, c), jnp.float32),
    )(p, wa, wb, b1row, w2, b2row)


def kernel(x, adj, batch_idx, num_graphs, eps0, W1_0, b1_0, g1_0, be1_0,
           W2_0, b2_0, gbn0, bebn0, eps1, W1_1, b1_1, g1_1, be1_1,
           W2_1, b2_1, gbn1, bebn1, Wfc1, bfc1, Wfc2, bfc2):
    n, d = x.shape
    hd = W1_0.shape[0]
    row = lambda v: v.reshape(1, -1)
    idx1d = batch_idx.astype(jnp.int32)
    scale0 = (1.0 + eps0).reshape(1, 1)
    scale1 = (1.0 + eps1).reshape(1, 1)

    t1, s1, s2 = _pass1(x, x.astype(jnp.bfloat16), adj, scale0,
                        W1_0.T, row(b1_0))
    u1, q1, q2 = _pass2(t1, s1, s2, row(g1_0), row(be1_0), W2_0.T, row(b2_0))
    h2f1, h2b1 = _pass3(u1, q1, q2, row(gbn0), row(bebn0))

    t2, s1b, s2b = _pass1(h2f1, h2b1, adj, scale1, W1_1.T, row(b1_1))
    u2, q1b, q2b = _pass2(t2, s1b, s2b, row(g1_1), row(be1_1),
                          W2_1.T, row(b2_1))
    h2f2, _h2b2 = _pass3(u2, q1b, q2b, row(gbn1), row(bebn1))

    idx_all = jnp.concatenate([idx1d, idx1d + G])
    p = _segsum_sc2(h2f1, h2f2, idx_all, jnp.zeros((2 * G, hd), jnp.float32))
    wa = Wfc1[:, :hd].T
    wb = Wfc1[:, hd:].T
    return _readout(p, wa, wb, row(bfc1), Wfc2.T, row(bfc2))


# speedup vs baseline: 1.0727x; 1.0727x over previous
import functools

import jax
import jax.numpy as jnp
from jax import lax
from jax.experimental import pallas as pl
from jax.experimental.pallas import tpu as pltpu
from jax.experimental.pallas import tpu_sc as plsc

G = 256

_TM1 = 200
_TM2 = 2000


def _bn_relu(v, s1, s2, g, be, n):
    mean = s1 * (1.0 / n)
    var = s2 * (1.0 / n) - mean * mean
    rstd = lax.rsqrt(var + 1e-5)
    return jnp.maximum(g * (v - mean) * rstd + be, 0.0)


def _layers_body(scale0_ref, scale1_ref, adj_ref, xb_ref, xt_ref,
                 w1t0_ref, b10_ref, g10_ref, be10_ref,
                 w2t0_ref, b20_ref, gbn0_ref, bebn0_ref,
                 w1t1_ref, b11_ref, g11_ref, be11_ref,
                 w2t1_ref, b21_ref, gbn1_ref, bebn1_ref,
                 h2f1_ref, h2f2_ref,
                 ws_sc, hb2_sc, s1_sc, s2_sc, q1_sc, q2_sc, *, n):
    i = pl.program_id(0)
    np0 = n // _TM1
    np2 = n // _TM2
    e1 = np0
    e2 = e1 + np2
    e3 = e2 + np2
    e4 = e3 + np0
    e5 = e4 + np2

    def agg_step(row0, hfull_bf16, htile, scale, w1t_ref, b1_ref, first):
        acc = jnp.dot(adj_ref[...].astype(jnp.bfloat16), hfull_bf16,
                      preferred_element_type=jnp.float32)
        out = scale * htile + acc
        t = jnp.dot(out, w1t_ref[...],
                    preferred_element_type=jnp.float32) + b1_ref[...]
        ws_sc[pl.ds(row0, _TM1), :] = t

        @pl.when(first)
        def _():
            s1_sc[...] = jnp.zeros_like(s1_sc)
            s2_sc[...] = jnp.zeros_like(s2_sc)

        s1_sc[...] += jnp.sum(t, axis=0, keepdims=True)
        s2_sc[...] += jnp.sum(t * t, axis=0, keepdims=True)

    def mlp2_step(j, g_ref, be_ref, w2t_ref, b2_ref):
        tt = ws_sc[pl.ds(j * _TM2, _TM2), :]
        h1 = _bn_relu(tt, s1_sc[...], s2_sc[...], g_ref[...], be_ref[...], n)
        u = jnp.dot(h1, w2t_ref[...],
                    preferred_element_type=jnp.float32) + b2_ref[...]
        ws_sc[pl.ds(j * _TM2, _TM2), :] = u

        @pl.when(j == 0)
        def _():
            q1_sc[...] = jnp.zeros_like(q1_sc)
            q2_sc[...] = jnp.zeros_like(q2_sc)

        q1_sc[...] += jnp.sum(u, axis=0, keepdims=True)
        q2_sc[...] += jnp.sum(u * u, axis=0, keepdims=True)

    @pl.when(i < e1)
    def _():
        agg_step(i * _TM1, xb_ref[...], xt_ref[...], scale0_ref[...],
                 w1t0_ref, b10_ref, i == 0)

    @pl.when((i >= e1) & (i < e2))
    def _():
        mlp2_step(i - e1, g10_ref, be10_ref, w2t0_ref, b20_ref)

    @pl.when((i >= e2) & (i < e3))
    def _():
        j = i - e2
        uu = ws_sc[pl.ds(j * _TM2, _TM2), :]
        h2 = _bn_relu(uu, q1_sc[...], q2_sc[...], gbn0_ref[...],
                      bebn0_ref[...], n)
        h2f1_ref[...] = h2
        hb2_sc[pl.ds(j * _TM2, _TM2), :] = h2.astype(jnp.bfloat16)

    @pl.when((i >= e3) & (i < e4))
    def _():
        r = (i - e3) * _TM1
        u_t = ws_sc[pl.ds(r, _TM1), :]
        h2t = _bn_relu(u_t, q1_sc[...], q2_sc[...], gbn0_ref[...],
                       bebn0_ref[...], n)
        agg_step(r, hb2_sc[...], h2t, scale1_ref[...],
                 w1t1_ref, b11_ref, i == e3)

    @pl.when((i >= e4) & (i < e5))
    def _():
        mlp2_step(i - e4, g11_ref, be11_ref, w2t1_ref, b21_ref)

    @pl.when(i >= e5)
    def _():
        j = i - e5
        uu = ws_sc[pl.ds(j * _TM2, _TM2), :]
        h2f2_ref[...] = _bn_relu(uu, q1_sc[...], q2_sc[...], gbn1_ref[...],
                                 bebn1_ref[...], n)


def _layers(x, xb, adj, scale0, scale1, wp0, wp1):
    n, hd = x.shape
    np0 = n // _TM1
    np2 = n // _TM2
    e1, e2, e3 = np0, np0 + np2, np0 + 2 * np2
    e4, e5 = np0 + 2 * np2 + np0, 2 * np0 + 3 * np2
    grid = (2 * np0 + 4 * np2,)

    def adj_map(i):
        return (jnp.where(i < e1, i,
                          jnp.where(i < e3, e1 - 1,
                                    jnp.where(i < e4, i - e3, e1 - 1))), 0)

    c00 = lambda i: (0, 0)
    small = pl.BlockSpec((1, hd), c00)
    wmat = pl.BlockSpec((hd, hd), c00)
    return pl.pallas_call(
        functools.partial(_layers_body, n=n),
        grid=grid,
        in_specs=[
            pl.BlockSpec((1, 1), c00),
            pl.BlockSpec((1, 1), c00),
            pl.BlockSpec((_TM1, adj.shape[1]), adj_map),
            pl.BlockSpec((n, hd), c00),
            pl.BlockSpec((_TM1, hd),
                         lambda i: (jnp.minimum(i, e1 - 1), 0)),
            wmat, small, small, small, wmat, small, small, small,
            wmat, small, small, small, wmat, small, small, small,
        ],
        out_specs=[
            pl.BlockSpec((_TM2, hd),
                         lambda i: (jnp.clip(i - e2, 0, np2 - 1), 0)),
            pl.BlockSpec((_TM2, hd),
                         lambda i: (jnp.clip(i - e5, 0, np2 - 1), 0)),
        ],
        out_shape=[
            jax.ShapeDtypeStruct((n, hd), jnp.float32),
            jax.ShapeDtypeStruct((n, hd), jnp.float32),
        ],
        scratch_shapes=[
            pltpu.VMEM((n, hd), jnp.float32),
            pltpu.VMEM((n, hd), jnp.bfloat16),
            pltpu.VMEM((1, hd), jnp.float32),
            pltpu.VMEM((1, hd), jnp.float32),
            pltpu.VMEM((1, hd), jnp.float32),
            pltpu.VMEM((1, hd), jnp.float32),
        ],
    )(scale0, scale1, adj, xb, x, *wp0, *wp1)


_BS = 80
_NW = 32


def _segsum_sc2(ha, hb, idx_cube, zeros2, nblk, kmax):
    n, hd = ha.shape
    tot = 2 * nblk
    mesh = plsc.VectorSubcoreMesh(core_axis_name="c", subcore_axis_name="s")

    @functools.partial(
        pl.kernel,
        out_type=jax.ShapeDtypeStruct((2, 2 * G, hd), jnp.float32),
        mesh=mesh,
        scratch_types=[
            pltpu.VMEM((kmax, _BS), jnp.int32),
            pltpu.VMEM((kmax, _BS, hd), jnp.float32),
            pltpu.VMEM_SHARED((2 * G, hd), jnp.float32),
            pltpu.SemaphoreType.DMA,
            pltpu.SemaphoreType.DMA,
        ],
    )
    def seg_kernel(ha_hbm, hb_hbm, idxc_hbm, zero_hbm, out_hbm,
                   idx_v, rows_v, shared, gsem, ssem):
        cid = lax.axis_index("c")
        sid = lax.axis_index("s")
        wid = cid * 16 + sid

        for k in range(kmax):
            b = wid + _NW * k
            lyr = b // nblk
            base = (b - lyr * nblk) * _BS

            @pl.when(b < tot)
            def _():
                @pl.when(lyr == 0)
                def _():
                    pltpu.async_copy(ha_hbm.at[pl.ds(base, _BS)],
                                     rows_v.at[k], gsem)

                @pl.when(lyr == 1)
                def _():
                    pltpu.async_copy(hb_hbm.at[pl.ds(base, _BS)],
                                     rows_v.at[k], gsem)

        pltpu.sync_copy(idxc_hbm.at[wid], idx_v)

        @pl.when(sid == 0)
        def _():
            pltpu.sync_copy(zero_hbm, shared)

        plsc.subcore_barrier()

        for k in range(kmax):
            b = wid + _NW * k

            @pl.when(b < tot)
            def _():
                pltpu.make_async_copy(ha_hbm.at[pl.ds(0, _BS)],
                                      rows_v.at[k], gsem).wait()

        descs = []
        for k in range(kmax):
            b = wid + _NW * k

            @pl.when(b < tot)
            def _():
                descs.append(pltpu.async_copy(
                    rows_v.at[k], shared.at[idx_v.at[k]], ssem, add=True))

        for k, d in enumerate(descs):
            b = wid + _NW * k

            @pl.when(b < tot)
            def _():
                d.wait()

        plsc.subcore_barrier()

        @pl.when(sid == 0)
        def _():
            pltpu.sync_copy(shared, out_hbm.at[cid])

    return seg_kernel(ha, hb, idx_cube, zeros2)


def _readout_body(p_ref, wa_ref, wb_ref, b1_ref, w2_ref, b2_ref, o_ref):
    acc = p_ref[0] + p_ref[1]
    seg1 = acc[:G]
    seg2 = acc[G:]
    o1 = (jnp.dot(seg1, wa_ref[...], preferred_element_type=jnp.float32)
          + jnp.dot(seg2, wb_ref[...], preferred_element_type=jnp.float32)
          + b1_ref[...])
    o1 = jnp.maximum(o1, 0.0)
    o_ref[...] = jnp.dot(o1, w2_ref[...],
                         preferred_element_type=jnp.float32) + b2_ref[...]


def _readout(p, wa, wb, b1row, w2, b2row):
    c = w2.shape[1]
    return pl.pallas_call(
        _readout_body,
        out_shape=jax.ShapeDtypeStruct((G, c), jnp.float32),
    )(p, wa, wb, b1row, w2, b2row)


def kernel(x, adj, batch_idx, num_graphs, eps0, W1_0, b1_0, g1_0, be1_0,
           W2_0, b2_0, gbn0, bebn0, eps1, W1_1, b1_1, g1_1, be1_1,
           W2_1, b2_1, gbn1, bebn1, Wfc1, bfc1, Wfc2, bfc2):
    n, d = x.shape
    hd = W1_0.shape[0]
    row = lambda v: v.reshape(1, -1)
    scale0 = (1.0 + eps0).reshape(1, 1)
    scale1 = (1.0 + eps1).reshape(1, 1)

    wp0 = (W1_0.T, row(b1_0), row(g1_0), row(be1_0),
           W2_0.T, row(b2_0), row(gbn0), row(bebn0))
    wp1 = (W1_1.T, row(b1_1), row(g1_1), row(be1_1),
           W2_1.T, row(b2_1), row(gbn1), row(bebn1))

    h2f1, h2f2 = _layers(x, x.astype(jnp.bfloat16), adj, scale0, scale1,
                         wp0, wp1)

    idx1d = batch_idx.astype(jnp.int32)
    nblk = n // _BS
    kmax = (2 * nblk + _NW - 1) // _NW
    idx_all = jnp.concatenate([idx1d, idx1d + G])
    pad = kmax * _NW * _BS - 2 * n
    idx_pad = jnp.concatenate([idx_all, jnp.zeros((pad,), jnp.int32)])
    idx_cube = idx_pad.reshape(kmax, _NW, _BS).transpose(1, 0, 2)

    p = _segsum_sc2(h2f1, h2f2, idx_cube,
                    jnp.zeros((2 * G, hd), jnp.float32), nblk, kmax)

    wa = Wfc1[:, :hd].T
    wb = Wfc1[:, hd:].T
    return _readout(p, wa, wb, row(bfc1), Wfc2.T, row(bfc2))


# speedup vs baseline: 1.1426x; 1.0652x over previous
import functools

import jax
import jax.numpy as jnp
from jax import lax
from jax.experimental import pallas as pl
from jax.experimental.pallas import tpu as pltpu
from jax.experimental.pallas import tpu_sc as plsc

G = 256

_TM1 = 400
_TM2 = 2000


def _bn_relu(v, s1, s2, g, be, n):
    mean = s1 * (1.0 / n)
    var = s2 * (1.0 / n) - mean * mean
    rstd = lax.rsqrt(var + 1e-5)
    return jnp.maximum(g * (v - mean) * rstd + be, 0.0)


def _layers_body(scale0_ref, scale1_ref, adj_ref, xb_ref,
                 w1t0_ref, b10_ref, g10_ref, be10_ref,
                 w2t0_ref, b20_ref, gbn0_ref, bebn0_ref,
                 w1t1_ref, b11_ref, g11_ref, be11_ref,
                 w2t1_ref, b21_ref, gbn1_ref, bebn1_ref,
                 h2f1_ref, h2f2_ref,
                 ws_sc, hb2_sc, s1_sc, s2_sc, q1_sc, q2_sc, *, n):
    i = pl.program_id(0)
    np0 = n // _TM1
    np2 = n // _TM2
    e1 = np0
    e2 = e1 + np2
    e3 = e2 + np2
    e4 = e3 + np0
    e5 = e4 + np2

    def agg_step(row0, hfull_bf16, htile, scale, w1t_ref, b1_ref, first):
        acc = jnp.dot(adj_ref[...].astype(jnp.bfloat16), hfull_bf16,
                      preferred_element_type=jnp.float32)
        out = scale * htile + acc
        t = jnp.dot(out, w1t_ref[...],
                    preferred_element_type=jnp.float32) + b1_ref[...]
        ws_sc[pl.ds(row0, _TM1), :] = t

        @pl.when(first)
        def _():
            s1_sc[...] = jnp.zeros_like(s1_sc)
            s2_sc[...] = jnp.zeros_like(s2_sc)

        s1_sc[...] += jnp.sum(t, axis=0, keepdims=True)
        s2_sc[...] += jnp.sum(t * t, axis=0, keepdims=True)

    def mlp2_step(j, g_ref, be_ref, w2t_ref, b2_ref):
        tt = ws_sc[pl.ds(j * _TM2, _TM2), :]
        h1 = _bn_relu(tt, s1_sc[...], s2_sc[...], g_ref[...], be_ref[...], n)
        u = jnp.dot(h1, w2t_ref[...],
                    preferred_element_type=jnp.float32) + b2_ref[...]
        ws_sc[pl.ds(j * _TM2, _TM2), :] = u

        @pl.when(j == 0)
        def _():
            q1_sc[...] = jnp.zeros_like(q1_sc)
            q2_sc[...] = jnp.zeros_like(q2_sc)

        q1_sc[...] += jnp.sum(u, axis=0, keepdims=True)
        q2_sc[...] += jnp.sum(u * u, axis=0, keepdims=True)

    @pl.when(i < e1)
    def _():
        r = i * _TM1
        xt = xb_ref[pl.ds(r, _TM1), :].astype(jnp.float32)
        agg_step(r, xb_ref[...], xt, scale0_ref[...],
                 w1t0_ref, b10_ref, i == 0)

    @pl.when((i >= e1) & (i < e2))
    def _():
        mlp2_step(i - e1, g10_ref, be10_ref, w2t0_ref, b20_ref)

    @pl.when((i >= e2) & (i < e3))
    def _():
        j = i - e2
        uu = ws_sc[pl.ds(j * _TM2, _TM2), :]
        h2 = _bn_relu(uu, q1_sc[...], q2_sc[...], gbn0_ref[...],
                      bebn0_ref[...], n)
        h2f1_ref[...] = h2
        hb2_sc[pl.ds(j * _TM2, _TM2), :] = h2.astype(jnp.bfloat16)

    @pl.when((i >= e3) & (i < e4))
    def _():
        r = (e4 - 1 - i) * _TM1
        u_t = ws_sc[pl.ds(r, _TM1), :]
        h2t = _bn_relu(u_t, q1_sc[...], q2_sc[...], gbn0_ref[...],
                       bebn0_ref[...], n)
        agg_step(r, hb2_sc[...], h2t, scale1_ref[...],
                 w1t1_ref, b11_ref, i == e3)

    @pl.when((i >= e4) & (i < e5))
    def _():
        mlp2_step(i - e4, g11_ref, be11_ref, w2t1_ref, b21_ref)

    @pl.when(i >= e5)
    def _():
        j = i - e5
        uu = ws_sc[pl.ds(j * _TM2, _TM2), :]
        h2f2_ref[...] = _bn_relu(uu, q1_sc[...], q2_sc[...], gbn1_ref[...],
                                 bebn1_ref[...], n)


def _layers(x, xb, adj, scale0, scale1, wp0, wp1):
    n, hd = x.shape
    np0 = n // _TM1
    np2 = n // _TM2
    e1, e2, e3 = np0, np0 + np2, np0 + 2 * np2
    e4, e5 = np0 + 2 * np2 + np0, 2 * np0 + 3 * np2
    grid = (2 * np0 + 4 * np2,)

    def adj_map(i):
        return (jnp.where(i < e1, i,
                          jnp.where(i < e3, e1 - 1,
                                    jnp.where(i < e4, e4 - 1 - i, 0))), 0)

    c00 = lambda i: (0, 0)
    small = pl.BlockSpec((1, hd), c00)
    wmat = pl.BlockSpec((hd, hd), c00)
    return pl.pallas_call(
        functools.partial(_layers_body, n=n),
        grid=grid,
        in_specs=[
            pl.BlockSpec((1, 1), c00),
            pl.BlockSpec((1, 1), c00),
            pl.BlockSpec((_TM1, adj.shape[1]), adj_map),
            pl.BlockSpec((n, hd), c00),
            wmat, small, small, small, wmat, small, small, small,
            wmat, small, small, small, wmat, small, small, small,
        ],
        out_specs=[
            pl.BlockSpec((_TM2, hd),
                         lambda i: (jnp.clip(i - e2, 0, np2 - 1), 0)),
            pl.BlockSpec((_TM2, hd),
                         lambda i: (jnp.clip(i - e5, 0, np2 - 1), 0)),
        ],
        out_shape=[
            jax.ShapeDtypeStruct((n, hd), jnp.float32),
            jax.ShapeDtypeStruct((n, hd), jnp.float32),
        ],
        scratch_shapes=[
            pltpu.VMEM((n, hd), jnp.float32),
            pltpu.VMEM((n, hd), jnp.bfloat16),
            pltpu.VMEM((1, hd), jnp.float32),
            pltpu.VMEM((1, hd), jnp.float32),
            pltpu.VMEM((1, hd), jnp.float32),
            pltpu.VMEM((1, hd), jnp.float32),
        ],
    )(scale0, scale1, adj, xb, *wp0, *wp1)


_BS = 80
_NW = 32


def _segsum_sc2(ha, hb, idx_cube, zeros2, nblk, kmax):
    n, hd = ha.shape
    tot = 2 * nblk
    mesh = plsc.VectorSubcoreMesh(core_axis_name="c", subcore_axis_name="s")

    @functools.partial(
        pl.kernel,
        out_type=jax.ShapeDtypeStruct((2, 2 * G, hd), jnp.float32),
        mesh=mesh,
        scratch_types=[
            pltpu.VMEM((kmax, _BS), jnp.int32),
            pltpu.VMEM((kmax, _BS, hd), jnp.float32),
            pltpu.VMEM_SHARED((2 * G, hd), jnp.float32),
            pltpu.SemaphoreType.DMA,
            pltpu.SemaphoreType.DMA,
        ],
    )
    def seg_kernel(ha_hbm, hb_hbm, idxc_hbm, zero_hbm, out_hbm,
                   idx_v, rows_v, shared, gsem, ssem):
        cid = lax.axis_index("c")
        sid = lax.axis_index("s")
        wid = cid * 16 + sid

        for k in range(kmax):
            b = wid + _NW * k
            lyr = b // nblk
            base = (b - lyr * nblk) * _BS

            @pl.when(b < tot)
            def _():
                @pl.when(lyr == 0)
                def _():
                    pltpu.async_copy(ha_hbm.at[pl.ds(base, _BS)],
                                     rows_v.at[k], gsem)

                @pl.when(lyr == 1)
                def _():
                    pltpu.async_copy(hb_hbm.at[pl.ds(base, _BS)],
                                     rows_v.at[k], gsem)

        pltpu.sync_copy(idxc_hbm.at[wid], idx_v)

        @pl.when(sid == 0)
        def _():
            pltpu.sync_copy(zero_hbm, shared)

        plsc.subcore_barrier()

        for k in range(kmax):
            b = wid + _NW * k

            @pl.when(b < tot)
            def _():
                pltpu.make_async_copy(ha_hbm.at[pl.ds(0, _BS)],
                                      rows_v.at[k], gsem).wait()

        descs = []
        for k in range(kmax):
            b = wid + _NW * k

            @pl.when(b < tot)
            def _():
                descs.append(pltpu.async_copy(
                    rows_v.at[k], shared.at[idx_v.at[k]], ssem, add=True))

        for k, d in enumerate(descs):
            b = wid + _NW * k

            @pl.when(b < tot)
            def _():
                d.wait()

        plsc.subcore_barrier()

        @pl.when(sid == 0)
        def _():
            pltpu.sync_copy(shared, out_hbm.at[cid])

    return seg_kernel(ha, hb, idx_cube, zeros2)


def _readout_body(p_ref, wa_ref, wb_ref, b1_ref, w2_ref, b2_ref, o_ref):
    acc = p_ref[0] + p_ref[1]
    seg1 = acc[:G]
    seg2 = acc[G:]
    o1 = (jnp.dot(seg1, wa_ref[...], preferred_element_type=jnp.float32)
          + jnp.dot(seg2, wb_ref[...], preferred_element_type=jnp.float32)
          + b1_ref[...])
    o1 = jnp.maximum(o1, 0.0)
    o_ref[...] = jnp.dot(o1, w2_ref[...],
                         preferred_element_type=jnp.float32) + b2_ref[...]


def _readout(p, wa, wb, b1row, w2, b2row):
    c = w2.shape[1]
    return pl.pallas_call(
        _readout_body,
        out_shape=jax.ShapeDtypeStruct((G, c), jnp.float32),
    )(p, wa, wb, b1row, w2, b2row)


def kernel(x, adj, batch_idx, num_graphs, eps0, W1_0, b1_0, g1_0, be1_0,
           W2_0, b2_0, gbn0, bebn0, eps1, W1_1, b1_1, g1_1, be1_1,
           W2_1, b2_1, gbn1, bebn1, Wfc1, bfc1, Wfc2, bfc2):
    n, d = x.shape
    hd = W1_0.shape[0]
    row = lambda v: v.reshape(1, -1)
    scale0 = (1.0 + eps0).reshape(1, 1)
    scale1 = (1.0 + eps1).reshape(1, 1)

    wp0 = (W1_0.T, row(b1_0), row(g1_0), row(be1_0),
           W2_0.T, row(b2_0), row(gbn0), row(bebn0))
    wp1 = (W1_1.T, row(b1_1), row(g1_1), row(be1_1),
           W2_1.T, row(b2_1), row(gbn1), row(bebn1))

    h2f1, h2f2 = _layers(x, x.astype(jnp.bfloat16), adj, scale0, scale1,
                         wp0, wp1)

    idx1d = batch_idx.astype(jnp.int32)
    nblk = n // _BS
    kmax = (2 * nblk + _NW - 1) // _NW
    idx_all = jnp.concatenate([idx1d, idx1d + G])
    pad = kmax * _NW * _BS - 2 * n
    idx_pad = jnp.concatenate([idx_all, jnp.zeros((pad,), jnp.int32)])
    idx_cube = idx_pad.reshape(kmax, _NW, _BS).transpose(1, 0, 2)

    p = _segsum_sc2(h2f1, h2f2, idx_cube,
                    jnp.zeros((2 * G, hd), jnp.float32), nblk, kmax)

    wa = Wfc1[:, :hd].T
    wb = Wfc1[:, hd:].T
    return _readout(p, wa, wb, row(bfc1), Wfc2.T, row(bfc2))
